# baseline (speedup 1.0000x reference)
import functools

import jax
import jax.numpy as jnp
from jax import lax
from jax.experimental import pallas as pl
from jax.experimental.pallas import tpu as pltpu
from jax.experimental.pallas import tpu_sc as plsc

NC = 2
NS = 16
L = 16
NW = NC * NS

D = 128
KN = 4
G_PAD = 102400
GB = 32
C_PER_W = G_PAD // NW
GPW = C_PER_W // GB
NB = 4
DV = D // L


def _sc_aggregate(
    mesh_hbm, iw_hbm, out_hbm,
    iw_all, r0, r1, r2, r3, a0, a1,
    sg0, sg1, sg2, sg3, so0, so1,
):
    wid = lax.axis_index("s") * NC + lax.axis_index("c")
    gbase = wid * GPW
    nbase = wid * C_PER_W

    rows = (r0, r1, r2, r3)
    accb = (a0, a1)
    sem_g = (sg0, sg1, sg2, sg3)
    sem_o = (so0, so1)

    pltpu.sync_copy(iw_hbm.at[pl.ds(gbase, GPW)], iw_all)

    def fire(g, slot):
        pltpu.async_copy(mesh_hbm.at[iw_all.at[g, 0]], rows[slot], sem_g[slot])

    def wait_g(slot):
        pltpu.make_async_copy(
            mesh_hbm.at[pl.ds(0, GB * KN)], rows[slot], sem_g[slot]
        ).wait()

    def start_out(ob, po):
        pltpu.async_copy(
            accb[po], out_hbm.at[pl.ds(nbase + ob * (2 * GB), 2 * GB)], sem_o[po]
        )

    def wait_out(po):
        pltpu.make_async_copy(
            accb[po], out_hbm.at[pl.ds(0, 2 * GB)], sem_o[po]
        ).wait()

    def compute(g, slot, po, half):
        rows_v = rows[slot]
        acc_v = accb[po]

        def node4(cg, carry2):
            wvec = lax.bitcast_convert_type(iw_all[g, 1, pl.ds(cg * 16, 16)], jnp.float32)
            for cc in range(4):
                c = cg * 4 + cc
                for j in range(DV):
                    acc = rows_v[c * KN, pl.ds(j * L, L)] * wvec[cc * KN]
                    for k in range(1, KN):
                        acc = acc + rows_v[c * KN + k, pl.ds(j * L, L)] * wvec[cc * KN + k]
                    acc_v[half * GB + c, pl.ds(j * L, L)] = acc
            return carry2

        lax.fori_loop(0, GB // 4, node4, 0)

    for s in range(NB - 1):
        fire(s, s)

    def body(i4, carry):
        for s in range(NB):
            g = i4 * NB + s
            fs = (s + NB - 1) % NB

            @pl.when(g + NB - 1 < GPW)
            def _():
                fire(g + NB - 1, fs)

            wait_g(s)
            po = (s // 2) % 2
            half = s % 2

            if half == 0:
                @pl.when(i4 > 0)
                def _():
                    wait_out(po)

            compute(g, s, po, half)

            if half == 1:
                start_out(i4 * 2 + s // 2, po)

        return carry

    lax.fori_loop(0, GPW // NB, body, 0)
    wait_out(0)
    wait_out(1)


_sc_call = pl.kernel(
    _sc_aggregate,
    out_type=jax.ShapeDtypeStruct((G_PAD, D), jnp.float32),
    mesh=plsc.VectorSubcoreMesh(
        core_axis_name="c", subcore_axis_name="s", num_cores=NC, num_subcores=NS
    ),
    scratch_types=[
        pltpu.VMEM((GPW, 2, GB * KN), jnp.int32),
        pltpu.VMEM((GB * KN, D), jnp.float32),
        pltpu.VMEM((GB * KN, D), jnp.float32),
        pltpu.VMEM((GB * KN, D), jnp.float32),
        pltpu.VMEM((GB * KN, D), jnp.float32),
        pltpu.VMEM((2 * GB, D), jnp.float32),
        pltpu.VMEM((2 * GB, D), jnp.float32),
        pltpu.SemaphoreType.DMA,
        pltpu.SemaphoreType.DMA,
        pltpu.SemaphoreType.DMA,
        pltpu.SemaphoreType.DMA,
        pltpu.SemaphoreType.DMA,
        pltpu.SemaphoreType.DMA,
    ],
)


R = 800


def _mlp_body(x_ref, w1_ref, b1_ref, g_ref, bb_ref, w2_ref, b2_ref, o_ref):
    x = x_ref[...]
    h = jnp.dot(x, w1_ref[...], preferred_element_type=jnp.float32) + b1_ref[...]
    mu = jnp.mean(h, axis=-1, keepdims=True)
    var = jnp.mean(jnp.square(h - mu), axis=-1, keepdims=True)
    h = (h - mu) * lax.rsqrt(var + 1e-5) * g_ref[...] + bb_ref[...]
    h = h * jax.nn.sigmoid(h)
    o_ref[...] = jnp.dot(h, w2_ref[...], preferred_element_type=jnp.float32) + b2_ref[...]


def _mlp_call(x, w1t, b1, g, bb, w2t, b2, num_grid):
    return pl.pallas_call(
        _mlp_body,
        grid=(num_grid // R,),
        in_specs=[
            pl.BlockSpec((R, D), lambda i: (i, 0)),
            pl.BlockSpec((D, D), lambda i: (0, 0)),
            pl.BlockSpec((1, D), lambda i: (0, 0)),
            pl.BlockSpec((1, D), lambda i: (0, 0)),
            pl.BlockSpec((1, D), lambda i: (0, 0)),
            pl.BlockSpec((D, D), lambda i: (0, 0)),
            pl.BlockSpec((1, D), lambda i: (0, 0)),
        ],
        out_specs=pl.BlockSpec((R, D), lambda i: (i, 0)),
        out_shape=jax.ShapeDtypeStruct((num_grid, D), jnp.float32),
    )(x, w1t, b1, g, bb, w2t, b2)


@jax.jit
def kernel(mesh_latent, m2g_indices, m2g_weights, W1, b1, ln_g, ln_b, W2, b2):
    b, m, d = mesh_latent.shape
    g, k = m2g_indices.shape
    mesh2d = mesh_latent.reshape(m * b, d)
    pad = G_PAD - g
    idx_flat = jnp.concatenate(
        [m2g_indices.astype(jnp.int32).reshape(-1), jnp.zeros((pad * k,), jnp.int32)]
    )
    w_flat = jnp.concatenate(
        [m2g_weights.reshape(-1), jnp.zeros((pad * k,), jnp.float32)]
    )
    n_groups_tot = G_PAD // GB
    iw = jnp.stack(
        [
            idx_flat.reshape(n_groups_tot, GB * KN),
            lax.bitcast_convert_type(w_flat, jnp.int32).reshape(n_groups_tot, GB * KN),
        ],
        axis=1,
    )
    grid_latent = _sc_call(mesh2d, iw)
    out = _mlp_call(
        grid_latent,
        W1.T,
        b1.reshape(1, d),
        ln_g.reshape(1, d),
        ln_b.reshape(1, d),
        W2.T,
        b2.reshape(1, d),
        g,
    )
    return out[None]


# speedup vs baseline: 1.7377x; 1.7377x over previous
import functools

import jax
import jax.numpy as jnp
from jax import lax
from jax.experimental import pallas as pl
from jax.experimental.pallas import tpu as pltpu
from jax.experimental.pallas import tpu_sc as plsc

NC = 2
NS = 16
L = 16
NW = NC * NS

D = 128
NUM_MESH = 10000
M_PAD = 10000
KN = 4
G_PAD = 102400
GB = 32
C_PER_W = G_PAD // NW
GPW = C_PER_W // GB
NB = 2
DV = D // L


def _sc_aggregate(
    mesh_hbm, iw_hbm, out_hbm,
    mesh_sh, iw_all, i0, i1, r0, r1, a0, a1,
    sg0, sg1, so0, so1,
):
    cid = lax.axis_index("c")
    sid = lax.axis_index("s")
    wid = sid * NC + cid
    gbase = wid * GPW
    nbase = wid * C_PER_W

    rows = (r0, r1)
    idxb = (i0, i1)
    accb = (a0, a1)
    sem_g = (sg0, sg1)
    sem_o = (so0, so1)

    mrows = 1000
    moff = pl.multiple_of(sid * mrows, 8)

    @pl.when(sid < 10)
    def _():
        pltpu.sync_copy(
            mesh_hbm.at[pl.ds(moff, mrows)], mesh_sh.at[pl.ds(moff, mrows)]
        )
    pltpu.sync_copy(iw_hbm.at[wid], iw_all)
    plsc.subcore_barrier()

    def fire(g, slot):
        for q in range(DV):
            pk = iw_all[g, pl.ds(q * 16, 16)]
            idxb[slot][pl.ds(q * 16, 16)] = pk & 0x3FFF
        pltpu.async_copy(mesh_sh.at[idxb[slot]], rows[slot], sem_g[slot])

    def wait_g(slot):
        pltpu.make_async_copy(
            mesh_hbm.at[pl.ds(0, GB * KN)], rows[slot], sem_g[slot]
        ).wait()

    HG = GB // 2

    def start_out(g, h):
        off = pl.multiple_of(nbase + g * GB + h * HG, HG)
        pltpu.async_copy(
            accb[h], out_hbm.at[pl.ds(off, HG)], sem_o[h]
        )

    def wait_out(h):
        pltpu.make_async_copy(
            accb[h], out_hbm.at[pl.ds(0, HG)], sem_o[h]
        ).wait()

    def compute_half(g, slot, h):
        rows_v = rows[slot]
        acc_v = accb[h]

        def node4(cg4, carry2):
            cg = h * (GB // 8) + cg4
            wvec = lax.bitcast_convert_type(
                iw_all[g, pl.ds(cg * 16, 16)] & jnp.int32(-16384), jnp.float32
            )
            for cc in range(4):
                c = cg * 4 + cc
                for j in range(DV):
                    acc = rows_v[c * KN, pl.ds(j * L, L)] * wvec[cc * KN]
                    for k in range(1, KN):
                        acc = acc + rows_v[c * KN + k, pl.ds(j * L, L)] * wvec[cc * KN + k]
                    acc_v[c - h * (GB // 2), pl.ds(j * L, L)] = acc
            return carry2

        lax.fori_loop(0, GB // 8, node4, 0)

    for s in range(NB - 1):
        fire(s, s)

    def body(i2, carry):
        for s in range(NB):
            g = i2 * NB + s
            fs = (s + NB - 1) % NB

            @pl.when(g + NB - 1 < GPW)
            def _():
                fire(g + NB - 1, fs)

            wait_g(s)

            for h in range(2):
                if s == 0:
                    @pl.when(i2 > 0)
                    def _():
                        wait_out(h)
                else:
                    wait_out(h)
                compute_half(g, s, h)
                start_out(g, h)

        return carry

    lax.fori_loop(0, GPW // NB, body, 0)
    wait_out(0)
    wait_out(1)


_sc_call = pl.kernel(
    _sc_aggregate,
    out_type=jax.ShapeDtypeStruct((G_PAD, D), jnp.float32),
    mesh=plsc.VectorSubcoreMesh(
        core_axis_name="c", subcore_axis_name="s", num_cores=NC, num_subcores=NS
    ),
    scratch_types=[
        pltpu.VMEM_SHARED((M_PAD, D), jnp.float32),
        pltpu.VMEM((GPW, GB * KN), jnp.int32),
        pltpu.VMEM((GB * KN,), jnp.int32),
        pltpu.VMEM((GB * KN,), jnp.int32),
        pltpu.VMEM((GB * KN, D), jnp.float32),
        pltpu.VMEM((GB * KN, D), jnp.float32),
        pltpu.VMEM((GB // 2, D), jnp.float32),
        pltpu.VMEM((GB // 2, D), jnp.float32),
        pltpu.SemaphoreType.DMA,
        pltpu.SemaphoreType.DMA,
        pltpu.SemaphoreType.DMA,
        pltpu.SemaphoreType.DMA,
    ],
)


R = 800


def _mlp_body(x_ref, w1_ref, b1_ref, g_ref, bb_ref, w2_ref, b2_ref, o_ref):
    x = x_ref[...]
    h = jnp.dot(x, w1_ref[...], preferred_element_type=jnp.float32) + b1_ref[...]
    mu = jnp.mean(h, axis=-1, keepdims=True)
    var = jnp.mean(jnp.square(h - mu), axis=-1, keepdims=True)
    h = (h - mu) * lax.rsqrt(var + 1e-5) * g_ref[...] + bb_ref[...]
    h = h * jax.nn.sigmoid(h)
    o_ref[...] = jnp.dot(h, w2_ref[...], preferred_element_type=jnp.float32) + b2_ref[...]


def _mlp_call(x, w1t, b1, g, bb, w2t, b2, num_grid):
    return pl.pallas_call(
        _mlp_body,
        grid=(num_grid // R,),
        in_specs=[
            pl.BlockSpec((R, D), lambda i: (i, 0)),
            pl.BlockSpec((D, D), lambda i: (0, 0)),
            pl.BlockSpec((1, D), lambda i: (0, 0)),
            pl.BlockSpec((1, D), lambda i: (0, 0)),
            pl.BlockSpec((1, D), lambda i: (0, 0)),
            pl.BlockSpec((D, D), lambda i: (0, 0)),
            pl.BlockSpec((1, D), lambda i: (0, 0)),
        ],
        out_specs=pl.BlockSpec((R, D), lambda i: (i, 0)),
        out_shape=jax.ShapeDtypeStruct((num_grid, D), jnp.float32),
    )(x, w1t, b1, g, bb, w2t, b2)


@jax.jit
def kernel(mesh_latent, m2g_indices, m2g_weights, W1, b1, ln_g, ln_b, W2, b2):
    b, m, d = mesh_latent.shape
    g, k = m2g_indices.shape
    mesh2d = mesh_latent.reshape(m * b, d)
    pad = G_PAD - g
    idx_flat = jnp.concatenate(
        [m2g_indices.astype(jnp.int32).reshape(-1), jnp.zeros((pad * k,), jnp.int32)]
    )
    w_flat = jnp.concatenate(
        [m2g_weights.reshape(-1), jnp.zeros((pad * k,), jnp.float32)]
    )
    n_groups_tot = G_PAD // GB
    w_bits = lax.bitcast_convert_type(w_flat, jnp.int32)
    w_hi = (w_bits + 0x2000) & jnp.int32(-16384)
    iw = (w_hi | idx_flat).reshape(NW, GPW, GB * KN)
    iw = jnp.pad(iw, ((0, 3 * NW), (0, 0), (0, 0)))
    grid_latent = _sc_call(mesh2d, iw)
    out = _mlp_call(
        grid_latent,
        W1.T,
        b1.reshape(1, d),
        ln_g.reshape(1, d),
        ln_b.reshape(1, d),
        W2.T,
        b2.reshape(1, d),
        g,
    )
    return out[None]


# speedup vs baseline: 3.3880x; 1.9498x over previous
import functools

import jax
import jax.numpy as jnp
from jax import lax
from jax.experimental import pallas as pl
from jax.experimental.pallas import tpu as pltpu
from jax.experimental.pallas import tpu_sc as plsc

NC = 2
NS = 16
L = 16
NW = NC * NS

D = 128
NUM_MESH = 10000
M_PAD = 10000
KN = 4
G_PAD = 102400
GB = 16
C_PER_W = G_PAD // NW
GPW = C_PER_W // GB
NB = 2
DV = D // L


def _sc_aggregate(
    mesh_hbm, iw_hbm, out_hbm,
    mesh_sh, iw_all, i0, i1, r0, r1, a0, a1,
    sg0, sg1, so0, so1,
):
    cid = lax.axis_index("c")
    sid = lax.axis_index("s")
    wid = sid * NC + cid
    gbase = wid * GPW
    nbase = wid * C_PER_W

    rows = (r0, r1)
    idxb = (i0, i1)
    accb = (a0, a1)
    sem_g = (sg0, sg1)
    sem_o = (so0, so1)

    mrows = 1000
    moff = pl.multiple_of(sid * mrows, 8)

    @pl.when(sid < 10)
    def _():
        pltpu.sync_copy(
            mesh_hbm.at[pl.ds(moff, mrows)], mesh_sh.at[pl.ds(moff, mrows)]
        )
    pltpu.sync_copy(iw_hbm.at[wid], iw_all)
    plsc.subcore_barrier()

    def fire(g, slot):
        for q in range(GB * KN // 16):
            pk = iw_all[g, pl.ds(q * 16, 16)]
            idxb[slot][pl.ds(q * 16, 16)] = pk & 0x3FFF
        pltpu.async_copy(mesh_sh.at[idxb[slot]], rows[slot], sem_g[slot])

    def wait_g(slot):
        pltpu.make_async_copy(
            mesh_hbm.at[pl.ds(0, GB * KN)], rows[slot], sem_g[slot]
        ).wait()

    def start_out(g, po):
        off = pl.multiple_of(nbase + g * GB, GB)
        pltpu.async_copy(
            accb[po], out_hbm.at[pl.ds(off, GB)], sem_o[po]
        )

    def wait_out(po):
        pltpu.make_async_copy(
            accb[po], out_hbm.at[pl.ds(0, GB)], sem_o[po]
        ).wait()

    def compute(g, slot, po):
        rows_v = rows[slot]
        acc_v = accb[po]

        def quad(cg, carry2):
            wq = [
                lax.bitcast_convert_type(
                    iw_all[g, pl.ds(k * GB + cg * 4, 16)] & jnp.int32(-16384),
                    jnp.float32,
                )
                for k in range(KN)
            ]
            for cc in range(4):
                c = cg * 4 + cc
                for j in range(DV):
                    acc = rows_v[c, pl.ds(j * L, L)] * wq[0][cc]
                    for k in range(1, KN):
                        acc = acc + rows_v[k * GB + c, pl.ds(j * L, L)] * wq[k][cc]
                    acc_v[c, pl.ds(j * L, L)] = acc
            return carry2

        lax.fori_loop(0, GB // 4, quad, 0)

    for s in range(NB - 1):
        fire(s, s)

    def body(i2, carry):
        for s in range(NB):
            g = i2 * NB + s
            fs = (s + NB - 1) % NB

            @pl.when(g + NB - 1 < GPW)
            def _():
                fire(g + NB - 1, fs)

            wait_g(s)

            @pl.when(i2 > 0)
            def _():
                wait_out(s)

            compute(g, s, s)
            start_out(g, s)

        return carry

    lax.fori_loop(0, GPW // NB, body, 0)
    wait_out(0)
    wait_out(1)


_sc_call = pl.kernel(
    _sc_aggregate,
    out_type=jax.ShapeDtypeStruct((G_PAD, D), jnp.float32),
    mesh=plsc.VectorSubcoreMesh(
        core_axis_name="c", subcore_axis_name="s", num_cores=NC, num_subcores=NS
    ),
    scratch_types=[
        pltpu.VMEM_SHARED((M_PAD, D), jnp.float32),
        pltpu.VMEM((GPW, GB * KN + 16), jnp.int32),
        pltpu.VMEM((GB * KN,), jnp.int32),
        pltpu.VMEM((GB * KN,), jnp.int32),
        pltpu.VMEM((GB * KN, D), jnp.float32),
        pltpu.VMEM((GB * KN, D), jnp.float32),
        pltpu.VMEM((GB, D), jnp.float32),
        pltpu.VMEM((GB, D), jnp.float32),
        pltpu.SemaphoreType.DMA,
        pltpu.SemaphoreType.DMA,
        pltpu.SemaphoreType.DMA,
        pltpu.SemaphoreType.DMA,
    ],
)


R = 800


def _mlp_body(x_ref, w1_ref, b1_ref, g_ref, bb_ref, w2_ref, b2_ref, o_ref):
    x = x_ref[...]
    h = jnp.dot(x, w1_ref[...], preferred_element_type=jnp.float32) + b1_ref[...]
    mu = jnp.mean(h, axis=-1, keepdims=True)
    var = jnp.mean(jnp.square(h - mu), axis=-1, keepdims=True)
    h = (h - mu) * lax.rsqrt(var + 1e-5) * g_ref[...] + bb_ref[...]
    h = h * jax.nn.sigmoid(h)
    o_ref[...] = jnp.dot(h, w2_ref[...], preferred_element_type=jnp.float32) + b2_ref[...]


def _mlp_call(x, w1t, b1, g, bb, w2t, b2, num_grid):
    return pl.pallas_call(
        _mlp_body,
        grid=(num_grid // R,),
        in_specs=[
            pl.BlockSpec((R, D), lambda i: (i, 0)),
            pl.BlockSpec((D, D), lambda i: (0, 0)),
            pl.BlockSpec((1, D), lambda i: (0, 0)),
            pl.BlockSpec((1, D), lambda i: (0, 0)),
            pl.BlockSpec((1, D), lambda i: (0, 0)),
            pl.BlockSpec((D, D), lambda i: (0, 0)),
            pl.BlockSpec((1, D), lambda i: (0, 0)),
        ],
        out_specs=pl.BlockSpec((R, D), lambda i: (i, 0)),
        out_shape=jax.ShapeDtypeStruct((num_grid, D), jnp.float32),
    )(x, w1t, b1, g, bb, w2t, b2)


@jax.jit
def kernel(mesh_latent, m2g_indices, m2g_weights, W1, b1, ln_g, ln_b, W2, b2):
    b, m, d = mesh_latent.shape
    g, k = m2g_indices.shape
    mesh2d = mesh_latent.reshape(m * b, d)
    pad = G_PAD - g
    w_bits = lax.bitcast_convert_type(m2g_weights, jnp.int32)
    w_hi = (w_bits + 0x2000) & jnp.int32(-16384)
    pk = w_hi | m2g_indices.astype(jnp.int32)
    pk = jnp.pad(pk, ((0, pad), (0, 0)))
    iw = (
        pk.T.reshape(KN, NW, GPW, GB)
        .transpose(1, 2, 0, 3)
        .reshape(NW, GPW, GB * KN)
    )
    iw = jnp.pad(iw, ((0, 0), (0, 0), (0, 16)))
    iw = jnp.pad(iw, ((0, 3 * NW), (0, 0), (0, 0)))
    grid_latent = _sc_call(mesh2d, iw)
    out = _mlp_call(
        grid_latent,
        W1.T,
        b1.reshape(1, d),
        ln_g.reshape(1, d),
        ln_b.reshape(1, d),
        W2.T,
        b2.reshape(1, d),
        g,
    )
    return out[None]


# speedup vs baseline: 4.2338x; 1.2496x over previous
import functools

import jax
import jax.numpy as jnp
from jax import lax
from jax.experimental import pallas as pl
from jax.experimental.pallas import tpu as pltpu
from jax.experimental.pallas import tpu_sc as plsc

NC = 2
NS = 16
L = 16
NW = NC * NS

D = 128
NUM_MESH = 10000
M_PAD = 10000
KN = 4
G_PAD = 102400
GB = 16
C_PER_W = G_PAD // NW
GPW = C_PER_W // GB
NB = 2
DV = D // L


def _sc_aggregate(
    mesh_hbm, iw_hbm, out_hbm,
    mesh_sh, iw_all, i0, i1, r0, r1, a0, a1,
    sg0, sg1, so0, so1,
):
    cid = lax.axis_index("c")
    sid = lax.axis_index("s")
    wid = sid * NC + cid
    gbase = wid * GPW
    nbase = wid * C_PER_W

    rows = (r0, r1)
    idxb = (i0, i1)
    accb = (a0, a1)
    sem_g = (sg0, sg1)
    sem_o = (so0, so1)

    mrows = 1000
    moff = pl.multiple_of(sid * mrows, 8)

    @pl.when(sid < 10)
    def _():
        pltpu.sync_copy(
            mesh_hbm.at[pl.ds(moff, mrows)], mesh_sh.at[pl.ds(moff, mrows)]
        )
    pltpu.sync_copy(iw_hbm.at[wid], iw_all)
    plsc.subcore_barrier()

    def fire(g, slot):
        for q in range(GB * KN // 16):
            pk = iw_all[g, pl.ds(q * 16, 16)]
            idxb[slot][pl.ds(q * 16, 16)] = pk & 0x3FFF
        pltpu.async_copy(mesh_sh.at[idxb[slot]], rows[slot], sem_g[slot])

    def wait_g(slot):
        pltpu.make_async_copy(
            mesh_hbm.at[pl.ds(0, GB * KN)], rows[slot], sem_g[slot]
        ).wait()

    def start_out(g, po):
        off = pl.multiple_of(nbase + g * GB, GB)
        pltpu.async_copy(
            accb[po], out_hbm.at[pl.ds(off, GB)], sem_o[po]
        )

    def wait_out(po):
        pltpu.make_async_copy(
            accb[po], out_hbm.at[pl.ds(0, GB)], sem_o[po]
        ).wait()

    def compute(g, slot, po):
        rows_v = rows[slot]
        acc_v = accb[po]

        def quad(cg, carry2):
            wq = [
                lax.bitcast_convert_type(
                    iw_all[g, pl.ds(k * GB + cg * 4, 16)] & jnp.int32(-16384),
                    jnp.float32,
                )
                for k in range(KN)
            ]
            for cc in range(4):
                c = cg * 4 + cc
                for j in range(DV):
                    acc = rows_v[c, pl.ds(j * L, L)] * wq[0][cc]
                    for k in range(1, KN):
                        acc = acc + rows_v[k * GB + c, pl.ds(j * L, L)] * wq[k][cc]
                    acc_v[c, pl.ds(j * L, L)] = acc
            return carry2

        lax.fori_loop(0, GB // 4, quad, 0)

    for s in range(NB - 1):
        fire(s, s)

    def body(i2, carry):
        for s in range(NB):
            g = i2 * NB + s
            fs = (s + NB - 1) % NB

            @pl.when(g + NB - 1 < GPW)
            def _():
                fire(g + NB - 1, fs)

            wait_g(s)

            @pl.when(i2 > 0)
            def _():
                wait_out(s)

            compute(g, s, s)
            start_out(g, s)

        return carry

    lax.fori_loop(0, GPW // NB, body, 0)
    wait_out(0)
    wait_out(1)


_sc_call = pl.kernel(
    _sc_aggregate,
    out_type=jax.ShapeDtypeStruct((G_PAD, D), jnp.float32),
    mesh=plsc.VectorSubcoreMesh(
        core_axis_name="c", subcore_axis_name="s", num_cores=NC, num_subcores=NS
    ),
    scratch_types=[
        pltpu.VMEM_SHARED((M_PAD, D), jnp.float32),
        pltpu.VMEM((GPW, GB * KN + 16), jnp.int32),
        pltpu.VMEM((GB * KN,), jnp.int32),
        pltpu.VMEM((GB * KN,), jnp.int32),
        pltpu.VMEM((GB * KN, D), jnp.float32),
        pltpu.VMEM((GB * KN, D), jnp.float32),
        pltpu.VMEM((GB, D), jnp.float32),
        pltpu.VMEM((GB, D), jnp.float32),
        pltpu.SemaphoreType.DMA,
        pltpu.SemaphoreType.DMA,
        pltpu.SemaphoreType.DMA,
        pltpu.SemaphoreType.DMA,
    ],
)


R = 2000


def _mlp_body(x_ref, w1_ref, b1_ref, g_ref, bb_ref, w2_ref, b2_ref, o_ref):
    x = x_ref[...]
    h = jnp.dot(x, w1_ref[...], preferred_element_type=jnp.float32) + b1_ref[...]
    mu = jnp.mean(h, axis=-1, keepdims=True)
    var = jnp.mean(jnp.square(h - mu), axis=-1, keepdims=True)
    h = (h - mu) * lax.rsqrt(var + 1e-5) * g_ref[...] + bb_ref[...]
    h = h * jax.nn.sigmoid(h)
    o_ref[...] = jnp.dot(h, w2_ref[...], preferred_element_type=jnp.float32) + b2_ref[...]


def _mlp_call(x, w1t, b1, g, bb, w2t, b2, num_grid):
    return pl.pallas_call(
        _mlp_body,
        grid=(num_grid // R,),
        in_specs=[
            pl.BlockSpec((R, D), lambda i: (i, 0)),
            pl.BlockSpec((D, D), lambda i: (0, 0)),
            pl.BlockSpec((1, D), lambda i: (0, 0)),
            pl.BlockSpec((1, D), lambda i: (0, 0)),
            pl.BlockSpec((1, D), lambda i: (0, 0)),
            pl.BlockSpec((D, D), lambda i: (0, 0)),
            pl.BlockSpec((1, D), lambda i: (0, 0)),
        ],
        out_specs=pl.BlockSpec((R, D), lambda i: (i, 0)),
        out_shape=jax.ShapeDtypeStruct((num_grid, D), jnp.float32),
    )(x, w1t, b1, g, bb, w2t, b2)


@jax.jit
def kernel(mesh_latent, m2g_indices, m2g_weights, W1, b1, ln_g, ln_b, W2, b2):
    b, m, d = mesh_latent.shape
    g, k = m2g_indices.shape
    mesh2d = mesh_latent.reshape(m * b, d)
    pad = G_PAD - g
    w_bits = lax.bitcast_convert_type(m2g_weights, jnp.int32)
    w_hi = (w_bits + 0x2000) & jnp.int32(-16384)
    pk = w_hi | m2g_indices.astype(jnp.int32)
    pk = jnp.pad(pk, ((0, pad), (0, 0)))
    iw = (
        pk.T.reshape(KN, NW, GPW, GB)
        .transpose(1, 2, 0, 3)
        .reshape(NW, GPW, GB * KN)
    )
    iw = jnp.pad(iw, ((0, 0), (0, 0), (0, 16)))
    grid_latent = _sc_call(mesh2d, iw)
    out = _mlp_call(
        grid_latent,
        W1.T,
        b1.reshape(1, d),
        ln_g.reshape(1, d),
        ln_b.reshape(1, d),
        W2.T,
        b2.reshape(1, d),
        g,
    )
    return out[None]


# speedup vs baseline: 4.5585x; 1.0767x over previous
import functools

import jax
import jax.numpy as jnp
from jax import lax
from jax.experimental import pallas as pl
from jax.experimental.pallas import tpu as pltpu
from jax.experimental.pallas import tpu_sc as plsc

NC = 2
NS = 16
L = 16
NW = NC * NS

D = 128
NUM_MESH = 10000
M_PAD = 10000
KN = 4
G_PAD = 102400
GB = 16
C_PER_W = G_PAD // NW
GPW = C_PER_W // GB
NB = 2
DV = D // L


def _sc_aggregate(
    mesh_hbm, iw_hbm, out_hbm,
    mesh_sh, iw_all, i0, i1, r0, r1, a0, a1,
    sg0, sg1, so0, so1,
):
    cid = lax.axis_index("c")
    sid = lax.axis_index("s")
    wid = sid * NC + cid
    gbase = wid * GPW
    nbase = wid * C_PER_W

    rows = (r0, r1)
    idxb = (i0, i1)
    accb = (a0, a1)
    sem_g = (sg0, sg1)
    sem_o = (so0, so1)

    mrows = 1000
    moff = pl.multiple_of(sid * mrows, 8)

    @pl.when(sid < 10)
    def _():
        pltpu.sync_copy(
            mesh_hbm.at[pl.ds(moff, mrows)], mesh_sh.at[pl.ds(moff, mrows)]
        )
    pltpu.sync_copy(iw_hbm.at[wid], iw_all)
    plsc.subcore_barrier()

    def fire(g, slot):
        for q in range(GB * KN // 16):
            pk = iw_all[g, pl.ds(q * 16, 16)]
            idxb[slot][pl.ds(q * 16, 16)] = pk & 0x3FFF
        pltpu.async_copy(mesh_sh.at[idxb[slot]], rows[slot], sem_g[slot])

    def wait_g(slot):
        pltpu.make_async_copy(
            mesh_hbm.at[pl.ds(0, GB * KN)], rows[slot], sem_g[slot]
        ).wait()

    def start_out(g, po):
        off = pl.multiple_of(nbase + g * GB, GB)
        pltpu.async_copy(
            accb[po], out_hbm.at[pl.ds(off, GB)], sem_o[po]
        )

    def wait_out(po):
        pltpu.make_async_copy(
            accb[po], out_hbm.at[pl.ds(0, GB)], sem_o[po]
        ).wait()

    def compute(g, slot, po):
        rows_v = rows[slot]
        acc_v = accb[po]

        def quad(cg, carry2):
            wq = [
                lax.bitcast_convert_type(
                    iw_all[g, pl.ds(k * GB + cg * 4, 16)] & jnp.int32(-16384),
                    jnp.float32,
                )
                for k in range(KN)
            ]
            for cc in range(4):
                c = cg * 4 + cc
                for j in range(DV):
                    acc = rows_v[c, pl.ds(j * L, L)] * wq[0][cc]
                    for k in range(1, KN):
                        acc = acc + rows_v[k * GB + c, pl.ds(j * L, L)] * wq[k][cc]
                    acc_v[c, pl.ds(j * L, L)] = acc
            return carry2

        lax.fori_loop(0, GB // 4, quad, 0)

    for s in range(NB - 1):
        fire(s, s)

    def body(ib, carry):
        for s in range(NB):
            g = ib * NB + s
            fs = (s + NB - 1) % NB

            @pl.when(g + NB - 1 < GPW)
            def _():
                fire(g + NB - 1, fs)

            wait_g(s)

            @pl.when(ib > 0)
            def _():
                wait_out(s)

            compute(g, s, s)
            start_out(g, s)

        return carry

    lax.fori_loop(0, GPW // NB, body, 0)
    for s in range(NB):
        wait_out(s)


_sc_call = pl.kernel(
    _sc_aggregate,
    out_type=jax.ShapeDtypeStruct((G_PAD, D), jnp.float32),
    mesh=plsc.VectorSubcoreMesh(
        core_axis_name="c", subcore_axis_name="s", num_cores=NC, num_subcores=NS
    ),
    scratch_types=[
        pltpu.VMEM_SHARED((M_PAD, D), jnp.float32),
        pltpu.VMEM((GPW, GB * KN + 16), jnp.int32),
        pltpu.VMEM((GB * KN,), jnp.int32),
        pltpu.VMEM((GB * KN,), jnp.int32),
        pltpu.VMEM((GB * KN, D), jnp.float32),
        pltpu.VMEM((GB * KN, D), jnp.float32),
        pltpu.VMEM((GB, D), jnp.float32),
        pltpu.VMEM((GB, D), jnp.float32),
        pltpu.SemaphoreType.DMA,
        pltpu.SemaphoreType.DMA,
        pltpu.SemaphoreType.DMA,
        pltpu.SemaphoreType.DMA,
    ],
)


R = 4000


def _mlp_body(x_ref, w1_ref, b1_ref, g_ref, bb_ref, w2_ref, b2_ref, o_ref):
    x = x_ref[...]
    h = jnp.dot(x, w1_ref[...], preferred_element_type=jnp.float32) + b1_ref[...]
    mu = jnp.mean(h, axis=-1, keepdims=True)
    var = jnp.mean(jnp.square(h - mu), axis=-1, keepdims=True)
    h = (h - mu) * lax.rsqrt(var + 1e-5) * g_ref[...] + bb_ref[...]
    h = h * jax.nn.sigmoid(h)
    o_ref[...] = jnp.dot(h, w2_ref[...], preferred_element_type=jnp.float32) + b2_ref[...]


def _mlp_call(x, w1t, b1, g, bb, w2t, b2, num_grid):
    return pl.pallas_call(
        _mlp_body,
        grid=(num_grid // R,),
        in_specs=[
            pl.BlockSpec((R, D), lambda i: (i, 0)),
            pl.BlockSpec((D, D), lambda i: (0, 0)),
            pl.BlockSpec((1, D), lambda i: (0, 0)),
            pl.BlockSpec((1, D), lambda i: (0, 0)),
            pl.BlockSpec((1, D), lambda i: (0, 0)),
            pl.BlockSpec((D, D), lambda i: (0, 0)),
            pl.BlockSpec((1, D), lambda i: (0, 0)),
        ],
        out_specs=pl.BlockSpec((R, D), lambda i: (i, 0)),
        out_shape=jax.ShapeDtypeStruct((num_grid, D), jnp.float32),
    )(x, w1t, b1, g, bb, w2t, b2)


@jax.jit
def kernel(mesh_latent, m2g_indices, m2g_weights, W1, b1, ln_g, ln_b, W2, b2):
    b, m, d = mesh_latent.shape
    g, k = m2g_indices.shape
    mesh2d = mesh_latent.reshape(m * b, d)
    pad = G_PAD - g
    w_bits = lax.bitcast_convert_type(m2g_weights, jnp.int32)
    w_hi = (w_bits + 0x2000) & jnp.int32(-16384)
    pk = w_hi | m2g_indices.astype(jnp.int32)
    pk = jnp.pad(pk, ((0, pad), (0, 0)))
    iw = (
        pk.T.reshape(KN, NW, GPW, GB)
        .transpose(1, 2, 0, 3)
        .reshape(NW, GPW, GB * KN)
    )
    iw = jnp.pad(iw, ((0, 0), (0, 0), (0, 16)))
    grid_latent = _sc_call(mesh2d, iw)
    out = _mlp_call(
        grid_latent,
        W1.T,
        b1.reshape(1, d),
        ln_g.reshape(1, d),
        ln_b.reshape(1, d),
        W2.T,
        b2.reshape(1, d),
        g,
    )
    return out[None]
